# baseline (speedup 1.0000x reference)
import jax
import jax.numpy as jnp
from jax.experimental import pallas as pl


def kernel(node_feat, edge_index, batch, W_rel, b_rel, W_root):
    src = edge_index[0]
    dst = edge_index[1]
    a = (node_feat @ W_rel.T).reshape(-1)
    agg_s = jax.ops.segment_sum(a[src], dst, num_segments=node_feat.shape[0])
    score = agg_s + b_rel[0] + (node_feat @ W_root.T).reshape(-1)
    k = min(2048, score.shape[0])
    top_scores, perm = jax.lax.top_k(score, k)
    x_out = node_feat[perm] * jnp.tanh(top_scores)[:, None]
    batch_out = batch[perm]
    return (x_out, batch_out)


# speedup vs baseline: 1.1495x; 1.1495x over previous
import functools

import jax
import jax.numpy as jnp
from jax import lax
from jax.experimental import pallas as pl
from jax.experimental.pallas import tpu as pltpu
from jax.experimental.pallas import tpu_sc as plsc

_N_PAD = 16384
_ROWS = 128
_COLS = 128
_K_OUT = 2048
_D = 128


def _topk_gate_body(score_ref, gate_ref, perm_ref):
    s = score_ref[...]
    li = (lax.broadcasted_iota(jnp.int32, (_ROWS, _COLS), 0) * _COLS
          + lax.broadcasted_iota(jnp.int32, (_ROWS, _COLS), 1))
    idx = li
    k = 2
    while k <= _N_PAD:
        j = k // 2
        while j >= 1:
            if j < _COLS:
                axis, sh = 1, j
            else:
                axis, sh = 0, j // _COLS
            low = (li & j) == 0
            ps = jnp.where(low, jnp.roll(s, -sh, axis), jnp.roll(s, sh, axis))
            pi = jnp.where(low, jnp.roll(idx, -sh, axis), jnp.roll(idx, sh, axis))
            larger_self = (s > ps) | ((s == ps) & (idx < pi))
            take_self = larger_self == (((li & k) == 0) == low)
            s = jnp.where(take_self, s, ps)
            idx = jnp.where(take_self, idx, pi)
            j //= 2
        k *= 2
    gate_ref[...] = jnp.tanh(s[:_K_OUT // _COLS, :])
    perm_ref[...] = idx[:_K_OUT // _COLS, :]


def _scale_body(rows_ref, gate_ref, out_ref):
    out_ref[...] = rows_ref[...] * gate_ref[...]


@functools.lru_cache(maxsize=1)
def _build_calls():
    topk_gate = pl.pallas_call(
        _topk_gate_body,
        out_shape=(
            jax.ShapeDtypeStruct((_K_OUT // _COLS, _COLS), jnp.float32),
            jax.ShapeDtypeStruct((_K_OUT // _COLS, _COLS), jnp.int32),
        ),
    )

    scale = pl.pallas_call(
        _scale_body,
        out_shape=jax.ShapeDtypeStruct((_K_OUT, _D), jnp.float32),
    )

    mesh = plsc.VectorSubcoreMesh(core_axis_name="c", subcore_axis_name="s")
    n_workers = 32
    per_w = _K_OUT // n_workers

    @functools.partial(
        pl.kernel,
        mesh=mesh,
        out_type=[
            jax.ShapeDtypeStruct((_K_OUT, _D), jnp.float32),
            jax.ShapeDtypeStruct((_K_OUT,), jnp.int32),
        ],
        scratch_types=[
            pltpu.VMEM((per_w,), jnp.int32),
            pltpu.VMEM((per_w, _D), jnp.float32),
            pltpu.VMEM((per_w,), jnp.int32),
            pltpu.SemaphoreType.DMA,
        ],
    )
    def gather_rows(feat_hbm, perm_hbm, batch_hbm, x_hbm, b_hbm,
                    idx_v, rows_v, bv_v, sem):
        wid = lax.axis_index("s") * 2 + lax.axis_index("c")
        base = wid * per_w
        pltpu.sync_copy(perm_hbm.at[pl.ds(base, per_w)], idx_v)
        pltpu.async_copy(feat_hbm.at[idx_v], rows_v, sem).wait()
        pltpu.async_copy(batch_hbm.at[idx_v], bv_v, sem).wait()
        pltpu.sync_copy(rows_v, x_hbm.at[pl.ds(base, per_w)])
        pltpu.sync_copy(bv_v, b_hbm.at[pl.ds(base, per_w)])

    return topk_gate, scale, gather_rows


def kernel(node_feat, edge_index, batch, W_rel, b_rel, W_root):
    topk_gate, scale, gather_rows = _build_calls()
    src = edge_index[0]
    dst = edge_index[1]
    msgs = node_feat[src]
    agg = jax.ops.segment_sum(msgs, dst, num_segments=node_feat.shape[0])
    score = (agg @ W_rel.T + b_rel + node_feat @ W_root.T).reshape(-1)
    spad = jnp.pad(score, (0, _N_PAD - score.shape[0]),
                   constant_values=-jnp.inf).reshape(_ROWS, _COLS)
    gate2d, perm2d = topk_gate(spad)
    perm = perm2d.reshape(_K_OUT)
    rows, batch_out = gather_rows(node_feat, perm, batch)
    x_out = scale(rows, gate2d.reshape(_K_OUT, 1))
    return (x_out, batch_out)


# speedup vs baseline: 1.1518x; 1.0020x over previous
import functools

import jax
import jax.numpy as jnp
from jax import lax
from jax.experimental import pallas as pl
from jax.experimental.pallas import tpu as pltpu
from jax.experimental.pallas import tpu_sc as plsc

_N_PAD = 16384
_ROWS = 128
_COLS = 128
_K_OUT = 2048
_D = 128


def _cmpx(s, idx, li, k, j):
    if j < _COLS:
        axis, sh = 1, j
    else:
        axis, sh = 0, j // _COLS
    low = (li & j) == 0
    ps = jnp.where(low, jnp.roll(s, -sh, axis), jnp.roll(s, sh, axis))
    pi = jnp.where(low, jnp.roll(idx, -sh, axis), jnp.roll(idx, sh, axis))
    larger_self = (s > ps) | ((s == ps) & (idx < pi))
    take_self = larger_self == (((li & k) == 0) == low)
    return jnp.where(take_self, s, ps), jnp.where(take_self, idx, pi)


def _lin_iota(rows):
    return (lax.broadcasted_iota(jnp.int32, (rows, _COLS), 0) * _COLS
            + lax.broadcasted_iota(jnp.int32, (rows, _COLS), 1))


def _topk_gate_body(score_ref, gate_ref, perm_ref):
    s = score_ref[...]
    li128 = _lin_iota(_ROWS)
    idx = li128
    s_a, i_a, l_a = s[:80], idx[:80], li128[:80]
    k = 2
    while k <= 2048:
        j = k // 2
        while j >= 1:
            s_a, i_a = _cmpx(s_a, i_a, l_a, k, j)
            j //= 2
        k *= 2
    s = jnp.concatenate([s_a, s[80:]], axis=0)
    idx = jnp.concatenate([i_a, idx[80:]], axis=0)
    rows = _ROWS
    for _ in range(3):
        li = _lin_iota(rows)
        s, idx = _cmpx(s, idx, li, 4096, 2048)
        keep = []
        for p in range(rows // 32):
            a = 32 * p if p % 2 == 0 else 32 * p + 16
            keep.append((a, a + 16))
        s = jnp.concatenate([s[a:b] for a, b in keep], axis=0)
        idx = jnp.concatenate([idx[a:b] for a, b in keep], axis=0)
        rows //= 2
        li = _lin_iota(rows)
        j = 1024
        while j >= 1:
            s, idx = _cmpx(s, idx, li, 2048, j)
            j //= 2
    gate_ref[...] = jnp.tanh(s)
    perm_ref[...] = idx


def _scale_body(rows_ref, gate_ref, out_ref):
    out_ref[...] = rows_ref[...] * gate_ref[...]


@functools.lru_cache(maxsize=1)
def _build_calls():
    topk_gate = pl.pallas_call(
        _topk_gate_body,
        out_shape=(
            jax.ShapeDtypeStruct((_K_OUT // _COLS, _COLS), jnp.float32),
            jax.ShapeDtypeStruct((_K_OUT // _COLS, _COLS), jnp.int32),
        ),
    )

    scale = pl.pallas_call(
        _scale_body,
        out_shape=jax.ShapeDtypeStruct((_K_OUT, _D), jnp.float32),
    )

    mesh = plsc.VectorSubcoreMesh(core_axis_name="c", subcore_axis_name="s")
    n_workers = 32
    per_w = _K_OUT // n_workers

    @functools.partial(
        pl.kernel,
        mesh=mesh,
        out_type=[
            jax.ShapeDtypeStruct((_K_OUT, _D), jnp.float32),
            jax.ShapeDtypeStruct((_K_OUT,), jnp.int32),
        ],
        scratch_types=[
            pltpu.VMEM((per_w,), jnp.int32),
            pltpu.VMEM((per_w, _D), jnp.float32),
            pltpu.VMEM((per_w,), jnp.int32),
            pltpu.SemaphoreType.DMA,
        ],
    )
    def gather_rows(feat_hbm, perm_hbm, batch_hbm, x_hbm, b_hbm,
                    idx_v, rows_v, bv_v, sem):
        wid = lax.axis_index("s") * 2 + lax.axis_index("c")
        base = wid * per_w
        pltpu.sync_copy(perm_hbm.at[pl.ds(base, per_w)], idx_v)
        pltpu.async_copy(feat_hbm.at[idx_v], rows_v, sem).wait()
        pltpu.async_copy(batch_hbm.at[idx_v], bv_v, sem).wait()
        pltpu.sync_copy(rows_v, x_hbm.at[pl.ds(base, per_w)])
        pltpu.sync_copy(bv_v, b_hbm.at[pl.ds(base, per_w)])

    return topk_gate, scale, gather_rows


def kernel(node_feat, edge_index, batch, W_rel, b_rel, W_root):
    topk_gate, scale, gather_rows = _build_calls()
    src = edge_index[0]
    dst = edge_index[1]
    msgs = node_feat[src]
    agg = jax.ops.segment_sum(msgs, dst, num_segments=node_feat.shape[0])
    score = (agg @ W_rel.T + b_rel + node_feat @ W_root.T).reshape(-1)
    spad = jnp.pad(score, (0, _N_PAD - score.shape[0]),
                   constant_values=-jnp.inf).reshape(_ROWS, _COLS)
    gate2d, perm2d = topk_gate(spad)
    perm = perm2d.reshape(_K_OUT)
    rows, batch_out = gather_rows(node_feat, perm, batch)
    x_out = scale(rows, gate2d.reshape(_K_OUT, 1))
    return (x_out, batch_out)
